# baseline (speedup 1.0000x reference)
import functools

import jax
import jax.numpy as jnp
from jax import lax
from jax.experimental import pallas as pl
from jax.experimental.pallas import tpu as pltpu
from jax.experimental.pallas import tpu_sc as plsc

N_DISTS = 100000
N_BITS = 16
HIDDEN = 2 * N_BITS
D = HIDDEN * N_BITS
B = 16384
NC, NS = 2, 16
NW = NC * NS
FPW = D // NW
SEG = 4096
NSEG = B // SEG
L = 16


def _make_gather():
    mesh = plsc.VectorSubcoreMesh(core_axis_name="c", subcore_axis_name="s")

    @functools.partial(
        pl.kernel,
        mesh=mesh,
        out_type=[
            jax.ShapeDtypeStruct((D, B), jnp.float32),
            jax.ShapeDtypeStruct((HIDDEN, B), jnp.float32),
        ],
        scratch_types=[
            pltpu.VMEM((B,), jnp.int32),
            pltpu.VMEM((N_DISTS,), jnp.float32),
            pltpu.VMEM((2, SEG), jnp.float32),
            pltpu.SemaphoreType.DMA,
            pltpu.SemaphoreType.DMA,
        ],
        compiler_params=pltpu.CompilerParams(needs_layout_passes=False),
    )
    def gather_kernel(x_hbm, wst_hbm, rst_hbm, w_out, r_out,
                      x_v, rowbuf, obuf, osem0, osem1):
        wid = lax.axis_index("s") * NC + lax.axis_index("c")
        pltpu.sync_copy(x_hbm, x_v)
        osems = (osem0, osem1)
        pending = [None, None]

        def do_row(row, src_t, out_t):
            pltpu.sync_copy(src_t.at[row], rowbuf)
            for seg in range(NSEG):
                slot = seg % 2
                if pending[slot] is not None:
                    pending[slot].wait()
                    pending[slot] = None

                def gbody(i, _):
                    off = i * (4 * L)
                    for j in range(4):
                        o = off + j * L
                        idx = x_v[pl.ds(seg * SEG + o, L)]
                        obuf[slot, pl.ds(o, L)] = plsc.load_gather(
                            rowbuf, [idx])
                    return 0

                lax.fori_loop(0, SEG // (4 * L), gbody, 0)
                pending[slot] = pltpu.async_copy(
                    obuf.at[slot], out_t.at[row, pl.ds(seg * SEG, SEG)],
                    osems[slot])

        for k in range(FPW):
            do_row(wid * FPW + k, wst_hbm, w_out)
        do_row(wid, rst_hbm, r_out)
        for slot in range(2):
            if pending[slot] is not None:
                pending[slot].wait()

    return gather_kernel


_gather = _make_gather()


def kernel(x, Ws, rs):
    Wt = Ws.transpose(1, 2, 0).reshape(D, N_DISTS)
    rt = rs.transpose(1, 0)
    OW, OR = _gather(x.astype(jnp.int32), Wt, rt)
    W = OW.reshape(HIDDEN, N_BITS, B).transpose(2, 0, 1)
    r = OR.transpose(1, 0)
    return (W, r)


# speedup vs baseline: 1.8683x; 1.8683x over previous
import functools

import jax
import jax.numpy as jnp
from jax import lax
from jax.experimental import pallas as pl
from jax.experimental.pallas import tpu as pltpu
from jax.experimental.pallas import tpu_sc as plsc

N_DISTS = 100000
N_BITS = 16
HIDDEN = 2 * N_BITS
D = HIDDEN * N_BITS
B = 16384
NC, NS = 2, 16
NW = NC * NS
FPW = D // NW
SEG = 4096
NSEG = B // SEG
L = 16


def _make_gather():
    mesh = plsc.VectorSubcoreMesh(core_axis_name="c", subcore_axis_name="s")

    @functools.partial(
        pl.kernel,
        mesh=mesh,
        out_type=[
            jax.ShapeDtypeStruct((D, B), jnp.float32),
            jax.ShapeDtypeStruct((HIDDEN, B), jnp.float32),
        ],
        scratch_types=[
            pltpu.VMEM((B,), jnp.int32),
            pltpu.VMEM((N_DISTS,), jnp.float32),
            pltpu.VMEM((2, SEG), jnp.float32),
            pltpu.SemaphoreType.DMA,
            pltpu.SemaphoreType.DMA,
        ],
        compiler_params=pltpu.CompilerParams(needs_layout_passes=False),
    )
    def gather_kernel(x_hbm, wst_hbm, rst_hbm, w_out, r_out,
                      x_v, rowbuf, obuf, osem0, osem1):
        wid = lax.axis_index("s") * NC + lax.axis_index("c")
        pltpu.sync_copy(x_hbm, x_v)
        osems = (osem0, osem1)
        pending = [None, None]

        def do_row(row, src_t, out_t):
            pltpu.sync_copy(src_t.at[row], rowbuf)
            for seg in range(NSEG):
                slot = seg % 2
                if pending[slot] is not None:
                    pending[slot].wait()
                    pending[slot] = None

                @plsc.parallel_loop(0, SEG, step=L, unroll=8)
                def gbody(i):
                    idx = x_v[pl.ds(seg * SEG + i, L)]
                    obuf[slot, pl.ds(i, L)] = plsc.load_gather(rowbuf, [idx])

                pending[slot] = pltpu.async_copy(
                    obuf.at[slot], out_t.at[row, pl.ds(seg * SEG, SEG)],
                    osems[slot])

        for k in range(FPW):
            do_row(wid * FPW + k, wst_hbm, w_out)
        do_row(wid, rst_hbm, r_out)
        for slot in range(2):
            if pending[slot] is not None:
                pending[slot].wait()

    return gather_kernel


_gather = _make_gather()


def kernel(x, Ws, rs):
    Wt = Ws.transpose(1, 2, 0).reshape(D, N_DISTS)
    rt = rs.transpose(1, 0)
    OW, OR = _gather(x.astype(jnp.int32), Wt, rt)
    W = OW.reshape(HIDDEN, N_BITS, B).transpose(2, 0, 1)
    r = OR.transpose(1, 0)
    return (W, r)
